# Initial kernel scaffold; baseline (speedup 1.0000x reference)
import functools

import jax
import jax.numpy as jnp
from jax import lax
from jax.experimental import pallas as pl
from jax.experimental.pallas import tpu as pltpu
from jax.experimental.pallas import tpu_sc as plsc

N_NODES = 10000
N_EDGES = 320000
D = 128

NC = 2
NS = 16
NW = NC * NS

CHUNK = 128
N_CHUNKS = N_EDGES // CHUNK
BASE_CHUNKS = N_CHUNKS // NW
EXTRA = N_CHUNKS - BASE_CHUNKS * NW

ROWS_PER_TILE = N_NODES // NS
ZROWS = 125

_mesh = plsc.VectorSubcoreMesh(core_axis_name="c", subcore_axis_name="s")


@functools.partial(
    pl.kernel,
    mesh=_mesh,
    out_type=jax.ShapeDtypeStruct((NC, N_NODES, D), jnp.float32),
    scratch_types=[
        pltpu.VMEM_SHARED((N_NODES, D), jnp.float32),
        pltpu.VMEM((ZROWS, D), jnp.float32),
        pltpu.VMEM((CHUNK,), jnp.int32),
        pltpu.VMEM((CHUNK, D), jnp.float32),
    ],
)
def _sc_scatter_add(x_hbm, idx_hbm, out_hbm, acc, zbuf, idxb, xb):
    c = lax.axis_index("c")
    s = lax.axis_index("s")
    wid = s * NC + c

    zero16 = jnp.zeros((16,), jnp.float32)

    def _zrow(r, _):
        for q in range(D // 16):
            zbuf[r, pl.ds(q * 16, 16)] = zero16
        return 0

    lax.fori_loop(0, ZROWS, _zrow, 0)
    for k in range(ROWS_PER_TILE // ZROWS):
        pltpu.sync_copy(zbuf, acc.at[pl.ds(s * ROWS_PER_TILE + k * ZROWS, ZROWS)])

    plsc.subcore_barrier()

    def _window(cg):
        off = cg * CHUNK
        pltpu.sync_copy(idx_hbm.at[pl.ds(off, CHUNK)], idxb)
        pltpu.sync_copy(x_hbm.at[pl.ds(off, CHUNK)], xb)
        pltpu.sync_copy(xb, acc.at[idxb], add=True)

    def _body(j, _):
        _window(wid + j * NW)
        return 0

    lax.fori_loop(0, BASE_CHUNKS, _body, 0)

    @pl.when(wid < EXTRA)
    def _tail():
        _window(BASE_CHUNKS * NW + wid)

    plsc.subcore_barrier()

    pltpu.sync_copy(
        acc.at[pl.ds(s * ROWS_PER_TILE, ROWS_PER_TILE)],
        out_hbm.at[c, pl.ds(s * ROWS_PER_TILE, ROWS_PER_TILE)],
    )


def _add_body(p_ref, o_ref):
    o_ref[...] = p_ref[0] + p_ref[1]


_ROWS_BLK = 1000


def _combine(partials):
    return pl.pallas_call(
        _add_body,
        grid=(N_NODES // _ROWS_BLK,),
        in_specs=[pl.BlockSpec((NC, _ROWS_BLK, D), lambda i: (0, i, 0))],
        out_specs=pl.BlockSpec((_ROWS_BLK, D), lambda i: (i, 0)),
        out_shape=jax.ShapeDtypeStruct((N_NODES, D), jnp.float32),
    )(partials)


def kernel(x, index):
    partials = _sc_scatter_add(x, index)
    return _combine(partials)


# speedup vs baseline: 4.5469x; 4.5469x over previous
import functools

import jax
import jax.numpy as jnp
from jax import lax
from jax.experimental import pallas as pl
from jax.experimental.pallas import tpu as pltpu
from jax.experimental.pallas import tpu_sc as plsc

N_NODES = 10000
N_PAD = 10240
N_EDGES = 320000
D = 128

NC = 2
NS = 16
NW = NC * NS

CHUNK = 128
N_CHUNKS = N_EDGES // CHUNK
BASE_CHUNKS = N_CHUNKS // NW
EXTRA = N_CHUNKS - BASE_CHUNKS * NW

ROWS_PER_TILE = N_PAD // NS
ZROWS = 128

_mesh = plsc.VectorSubcoreMesh(core_axis_name="c", subcore_axis_name="s")


@functools.partial(
    pl.kernel,
    mesh=_mesh,
    out_type=jax.ShapeDtypeStruct((NC, N_PAD, D), jnp.float32),
    scratch_types=[
        pltpu.VMEM_SHARED((N_PAD, D), jnp.float32),
        pltpu.VMEM((ZROWS, D), jnp.float32),
        pltpu.VMEM((CHUNK,), jnp.int32),
        pltpu.VMEM((CHUNK, D), jnp.float32),
    ],
)
def _sc_scatter_add(x_hbm, idx_hbm, out_hbm, acc, zbuf, idxb, xb):
    c = lax.axis_index("c")
    s = lax.axis_index("s")
    wid = s * NC + c

    zero16 = jnp.zeros((16,), jnp.float32)

    def _zrow(r, _):
        for q in range(D // 16):
            zbuf[r, pl.ds(q * 16, 16)] = zero16
        return 0

    lax.fori_loop(0, ZROWS, _zrow, 0)
    for k in range(ROWS_PER_TILE // ZROWS):
        pltpu.sync_copy(zbuf, acc.at[pl.ds(s * ROWS_PER_TILE + k * ZROWS, ZROWS)])

    plsc.subcore_barrier()

    def _window(cg):
        off = cg * CHUNK
        pltpu.sync_copy(idx_hbm.at[pl.ds(off, CHUNK)], idxb)
        pltpu.sync_copy(x_hbm.at[pl.ds(off, CHUNK)], xb)
        pltpu.sync_copy(xb, acc.at[idxb], add=True)

    def _body(j, _):
        _window(wid + j * NW)
        return 0

    lax.fori_loop(0, BASE_CHUNKS, _body, 0)

    @pl.when(wid < EXTRA)
    def _tail():
        _window(BASE_CHUNKS * NW + wid)

    plsc.subcore_barrier()

    pltpu.sync_copy(
        acc.at[pl.ds(s * ROWS_PER_TILE, ROWS_PER_TILE)],
        out_hbm.at[c, pl.ds(s * ROWS_PER_TILE, ROWS_PER_TILE)],
    )


def _add_body(p_ref, o_ref):
    o_ref[...] = p_ref[0] + p_ref[1]


_ROWS_BLK = 1000


def _combine(partials):
    return pl.pallas_call(
        _add_body,
        grid=(N_NODES // _ROWS_BLK,),
        in_specs=[pl.BlockSpec((NC, _ROWS_BLK, D), lambda i: (0, i, 0))],
        out_specs=pl.BlockSpec((_ROWS_BLK, D), lambda i: (i, 0)),
        out_shape=jax.ShapeDtypeStruct((N_NODES, D), jnp.float32),
    )(partials)


def kernel(x, index):
    partials = _sc_scatter_add(x, index)
    return _combine(partials)


# speedup vs baseline: 8.8256x; 1.9410x over previous
import functools

import jax
import jax.numpy as jnp
from jax import lax
from jax.experimental import pallas as pl
from jax.experimental.pallas import tpu as pltpu
from jax.experimental.pallas import tpu_sc as plsc

N_NODES = 10000
N_EDGES = 320000
D = 128

NC = 2
NS = 16
NW = NC * NS

CHUNK = 128
N_CHUNKS = N_EDGES // CHUNK
BASE_CHUNKS = N_CHUNKS // NW
EXTRA = N_CHUNKS - BASE_CHUNKS * NW

NBUF = 3
GROUPS = BASE_CHUNKS // NBUF

ROWS_MAIN = 624

_mesh = plsc.VectorSubcoreMesh(core_axis_name="c", subcore_axis_name="s")


@functools.partial(
    pl.kernel,
    mesh=_mesh,
    out_type=jax.ShapeDtypeStruct((NC, N_NODES, D), jnp.float32),
    scratch_types=[
        pltpu.VMEM_SHARED((N_NODES, D), jnp.float32),
        pltpu.VMEM((NBUF, CHUNK), jnp.int32),
        pltpu.VMEM((NBUF, CHUNK, D), jnp.float32),
        pltpu.SemaphoreType.DMA((NBUF,)),
        pltpu.SemaphoreType.DMA((NBUF,)),
    ],
)
def _sc_scatter_add(x_hbm, idx_hbm, out_hbm, acc, idxr, xb, ld, sc):
    c = lax.axis_index("c")
    s = lax.axis_index("s")
    wid = s * NC + c

    zero16 = jnp.zeros((16,), jnp.float32)

    def _zrow(r, _):
        for q in range(D // 16):
            xb[0, r, pl.ds(q * 16, 16)] = zero16
        return 0

    lax.fori_loop(0, CHUNK, _zrow, 0)
    row0 = s * ROWS_MAIN
    for k in range(ROWS_MAIN // CHUNK):
        pltpu.sync_copy(xb.at[0], acc.at[pl.ds(row0 + k * CHUNK, CHUNK)])
    pltpu.sync_copy(xb.at[0, pl.ds(0, ROWS_MAIN % CHUNK)],
                    acc.at[pl.ds(row0 + (ROWS_MAIN // CHUNK) * CHUNK,
                                 ROWS_MAIN % CHUNK)])

    @pl.when(s < 2)
    def _ztail():
        pltpu.sync_copy(xb.at[0, pl.ds(0, 8)],
                        acc.at[pl.ds(NS * ROWS_MAIN + s * 8, 8)])

    plsc.subcore_barrier()

    def fire_load(b, cg):
        off = cg * CHUNK
        pltpu.async_copy(idx_hbm.at[pl.ds(off, CHUNK)], idxr.at[b], ld.at[b])
        pltpu.async_copy(x_hbm.at[pl.ds(off, CHUNK)], xb.at[b], ld.at[b])

    def wait_load(b):
        pltpu.make_async_copy(
            idx_hbm.at[pl.ds(0, CHUNK)], idxr.at[b], ld.at[b]).wait()
        pltpu.make_async_copy(
            x_hbm.at[pl.ds(0, CHUNK)], xb.at[b], ld.at[b]).wait()

    def fire_sc(b):
        pltpu.async_copy(xb.at[b], acc.at[idxr.at[b]], sc.at[b], add=True)

    def wait_sc(b):
        pltpu.make_async_copy(xb.at[b], acc.at[idxr.at[b]], sc.at[b]).wait()

    for b in range(NBUF):
        fire_load(b, wid + b * NW)

    def _group(g, _):
        for b in range(NBUF):
            j = g * NBUF + b
            wait_load(b)
            fire_sc(b)
            wait_sc(b)

            @pl.when(g < GROUPS - 1)
            def _():
                fire_load(b, wid + (j + NBUF) * NW)

        return 0

    lax.fori_loop(0, GROUPS, _group, 0)

    @pl.when(wid < EXTRA)
    def _tail():
        off = (BASE_CHUNKS * NW + wid) * CHUNK
        pltpu.sync_copy(idx_hbm.at[pl.ds(off, CHUNK)], idxr.at[0])
        pltpu.sync_copy(x_hbm.at[pl.ds(off, CHUNK)], xb.at[0])
        pltpu.sync_copy(xb.at[0], acc.at[idxr.at[0]], add=True)

    plsc.subcore_barrier()

    pltpu.sync_copy(
        acc.at[pl.ds(s * ROWS_MAIN, ROWS_MAIN)],
        out_hbm.at[c, pl.ds(s * ROWS_MAIN, ROWS_MAIN)],
    )

    @pl.when(s < 2)
    def _wtail():
        pltpu.sync_copy(
            acc.at[pl.ds(NS * ROWS_MAIN + s * 8, 8)],
            out_hbm.at[c, pl.ds(NS * ROWS_MAIN + s * 8, 8)],
        )


def _add_body(p_ref, o_ref):
    o_ref[...] = p_ref[0] + p_ref[1]


_ROWS_BLK = 1000


def _combine(partials):
    return pl.pallas_call(
        _add_body,
        grid=(N_NODES // _ROWS_BLK,),
        in_specs=[pl.BlockSpec((NC, _ROWS_BLK, D), lambda i: (0, i, 0))],
        out_specs=pl.BlockSpec((_ROWS_BLK, D), lambda i: (i, 0)),
        out_shape=jax.ShapeDtypeStruct((N_NODES, D), jnp.float32),
    )(partials)


def kernel(x, index):
    partials = _sc_scatter_add(x, index)
    return _combine(partials)


# speedup vs baseline: 8.9929x; 1.0190x over previous
import functools

import jax
import jax.numpy as jnp
from jax import lax
from jax.experimental import pallas as pl
from jax.experimental.pallas import tpu as pltpu
from jax.experimental.pallas import tpu_sc as plsc

N_NODES = 10000
N_EDGES = 320000
D = 128

NC = 2
NS = 16
NW = NC * NS

CHUNK = 128
N_CHUNKS = N_EDGES // CHUNK
BASE_CHUNKS = N_CHUNKS // NW
EXTRA = N_CHUNKS - BASE_CHUNKS * NW

NBUF = 3
GROUPS = BASE_CHUNKS // NBUF

ROWS_MAIN = 624

_mesh = plsc.VectorSubcoreMesh(core_axis_name="c", subcore_axis_name="s")


@functools.partial(
    pl.kernel,
    mesh=_mesh,
    out_type=jax.ShapeDtypeStruct((NC, N_NODES, D), jnp.float32),
    scratch_types=[
        pltpu.VMEM_SHARED((N_NODES, D), jnp.float32),
        pltpu.VMEM((NBUF, CHUNK), jnp.int32),
        pltpu.VMEM((NBUF, CHUNK, D), jnp.float32),
        pltpu.SemaphoreType.DMA((NBUF,)),
        pltpu.SemaphoreType.DMA((NBUF,)),
    ],
)
def _sc_scatter_add(x_hbm, idx_hbm, out_hbm, acc, idxr, xb, ld, sc):
    c = lax.axis_index("c")
    s = lax.axis_index("s")
    wid = s * NC + c

    zero16 = jnp.zeros((16,), jnp.float32)

    def _zrow(r, _):
        for q in range(D // 16):
            xb[0, r, pl.ds(q * 16, 16)] = zero16
        return 0

    lax.fori_loop(0, CHUNK, _zrow, 0)
    row0 = s * ROWS_MAIN
    for k in range(ROWS_MAIN // CHUNK):
        pltpu.sync_copy(xb.at[0], acc.at[pl.ds(row0 + k * CHUNK, CHUNK)])
    pltpu.sync_copy(xb.at[0, pl.ds(0, ROWS_MAIN % CHUNK)],
                    acc.at[pl.ds(row0 + (ROWS_MAIN // CHUNK) * CHUNK,
                                 ROWS_MAIN % CHUNK)])

    @pl.when(s < 2)
    def _ztail():
        pltpu.sync_copy(xb.at[0, pl.ds(0, 8)],
                        acc.at[pl.ds(NS * ROWS_MAIN + s * 8, 8)])

    plsc.subcore_barrier()

    def fire_load(b, cg):
        off = cg * CHUNK
        pltpu.async_copy(idx_hbm.at[pl.ds(off, CHUNK)], idxr.at[b], ld.at[b])
        pltpu.async_copy(x_hbm.at[pl.ds(off, CHUNK)], xb.at[b], ld.at[b])

    def wait_load(b):
        pltpu.make_async_copy(
            idx_hbm.at[pl.ds(0, CHUNK)], idxr.at[b], ld.at[b]).wait()
        pltpu.make_async_copy(
            x_hbm.at[pl.ds(0, CHUNK)], xb.at[b], ld.at[b]).wait()

    def fire_sc(b):
        pltpu.async_copy(xb.at[b], acc.at[idxr.at[b]], sc.at[b], add=True)

    def wait_sc(b):
        pltpu.make_async_copy(xb.at[b], acc.at[idxr.at[b]], sc.at[b]).wait()

    fire_load(0, wid + 0 * NW)
    fire_load(1, wid + 1 * NW)

    fire_load(2, wid + 2 * NW)
    wait_load(0)
    fire_sc(0)

    def _group(g, _):
        for b in range(NBUF):
            j = 1 + g * NBUF + b
            wait_sc(b)
            fire_load(b, wid + (j + 2) * NW)
            wait_load((b + 1) % NBUF)
            fire_sc((b + 1) % NBUF)
        return 0

    lax.fori_loop(0, 25, _group, 0)

    wait_sc(0)
    wait_load(1)
    fire_sc(1)
    wait_sc(1)
    wait_load(2)
    fire_sc(2)
    wait_sc(2)

    @pl.when(wid < EXTRA)
    def _tail():
        off = (BASE_CHUNKS * NW + wid) * CHUNK
        pltpu.sync_copy(idx_hbm.at[pl.ds(off, CHUNK)], idxr.at[0])
        pltpu.sync_copy(x_hbm.at[pl.ds(off, CHUNK)], xb.at[0])
        pltpu.sync_copy(xb.at[0], acc.at[idxr.at[0]], add=True)

    plsc.subcore_barrier()

    pltpu.sync_copy(
        acc.at[pl.ds(s * ROWS_MAIN, ROWS_MAIN)],
        out_hbm.at[c, pl.ds(s * ROWS_MAIN, ROWS_MAIN)],
    )

    @pl.when(s < 2)
    def _wtail():
        pltpu.sync_copy(
            acc.at[pl.ds(NS * ROWS_MAIN + s * 8, 8)],
            out_hbm.at[c, pl.ds(NS * ROWS_MAIN + s * 8, 8)],
        )


def _add_body(p_ref, o_ref):
    o_ref[...] = p_ref[0] + p_ref[1]


_ROWS_BLK = 2000


def _combine(partials):
    return pl.pallas_call(
        _add_body,
        grid=(N_NODES // _ROWS_BLK,),
        in_specs=[pl.BlockSpec((NC, _ROWS_BLK, D), lambda i: (0, i, 0))],
        out_specs=pl.BlockSpec((_ROWS_BLK, D), lambda i: (i, 0)),
        out_shape=jax.ShapeDtypeStruct((N_NODES, D), jnp.float32),
    )(partials)


def kernel(x, index):
    partials = _sc_scatter_add(x, index)
    return _combine(partials)
